# Initial kernel scaffold; baseline (speedup 1.0000x reference)
import jax
import jax.numpy as jnp
from jax.experimental import pallas as pl


def kernel(src, s_SPD, W1, a, W2):
    raise NotImplementedError("write your pallas kernel here")


# speedup vs baseline: 3.0362x; 3.0362x over previous
import jax
import jax.numpy as jnp
from jax.experimental import pallas as pl

_B, _N, _C = 128, 25, 128
_J = 25
_HID = 32
_K = 8
_E = _J * _J


def _edge_attn_body(spd_ref, src_ref, w1t_ref, a_ref, w2_ref, out_ref):
    spd = spd_ref[...]
    n_iota = jax.lax.broadcasted_iota(jnp.int32, (_E, _N), 1)
    d = jnp.zeros((_E, _N), dtype=jnp.float32)
    for k in range(_K - 1):
        ends_k = spd[:, k][:, None]
        heads_k = spd[:, k][:, None]
        d = d + (ends_k == n_iota).astype(jnp.float32)
        d = d - (heads_k == n_iota).astype(jnp.float32)

    src_b = src_ref[0]
    p = jnp.dot(src_b, w1t_ref[...], preferred_element_type=jnp.float32)
    h = jnp.dot(d, p, preferred_element_type=jnp.float32)
    alpha = a_ref[0, 0]
    h = jnp.maximum(h, 0.0) + alpha * jnp.minimum(h, 0.0)
    o = jnp.sum(h * w2_ref[0][None, :], axis=1)
    out_ref[...] = o[None, None, :]


def kernel(src, s_SPD, W1, a, W2):
    spd = s_SPD.reshape(_E, _K)
    w1t = W1.T
    a2 = a.reshape(1, 1)
    out = pl.pallas_call(
        _edge_attn_body,
        grid=(_B,),
        in_specs=[
            pl.BlockSpec((_E, _K), lambda b: (0, 0)),
            pl.BlockSpec((1, _N, _C), lambda b: (b, 0, 0)),
            pl.BlockSpec((_C, _HID), lambda b: (0, 0)),
            pl.BlockSpec((1, 1), lambda b: (0, 0)),
            pl.BlockSpec((1, _HID), lambda b: (0, 0)),
        ],
        out_specs=pl.BlockSpec((1, 1, _E), lambda b: (b, 0, 0)),
        out_shape=jax.ShapeDtypeStruct((_B, 1, _E), jnp.float32),
    )(spd, src, w1t, a2, W2)
    return out.reshape(_B, _J, _J, 1)


# speedup vs baseline: 4.4998x; 1.4821x over previous
import jax
import jax.numpy as jnp
from jax.experimental import pallas as pl
from jax.experimental.pallas import tpu as pltpu

_B, _N, _C = 128, 25, 128
_J = 25
_HID = 32
_K = 8
_E = _J * _J


def _edge_attn_body(spd_ref, src_ref, w1t_ref, a_ref, w2_ref, out_ref, d_ref):
    @pl.when(pl.program_id(0) == 0)
    def _build_d():
        spd = spd_ref[...]
        n_iota = jax.lax.broadcasted_iota(jnp.int32, (_E, _N), 1)
        d = jnp.zeros((_E, _N), dtype=jnp.float32)
        for k in range(_K - 1):
            ends_k = spd[:, k][:, None]
            heads_k = spd[:, k][:, None]
            d = d + (ends_k == n_iota).astype(jnp.float32)
            d = d - (heads_k == n_iota).astype(jnp.float32)
        d_ref[...] = d

    d = d_ref[...]
    src_b = src_ref[0]
    p = jnp.dot(src_b, w1t_ref[...], preferred_element_type=jnp.float32)
    h = jnp.dot(d, p, preferred_element_type=jnp.float32)
    alpha = a_ref[0, 0]
    h = jnp.maximum(h, 0.0) + alpha * jnp.minimum(h, 0.0)
    o = jnp.sum(h * w2_ref[0][None, :], axis=1)
    out_ref[...] = o[None, None, :]


def kernel(src, s_SPD, W1, a, W2):
    spd = s_SPD.reshape(_E, _K)
    w1t = W1.T
    a2 = a.reshape(1, 1)
    out = pl.pallas_call(
        _edge_attn_body,
        grid=(_B,),
        in_specs=[
            pl.BlockSpec((_E, _K), lambda b: (0, 0)),
            pl.BlockSpec((1, _N, _C), lambda b: (b, 0, 0)),
            pl.BlockSpec((_C, _HID), lambda b: (0, 0)),
            pl.BlockSpec((1, 1), lambda b: (0, 0)),
            pl.BlockSpec((1, _HID), lambda b: (0, 0)),
        ],
        out_specs=pl.BlockSpec((1, 1, _E), lambda b: (b, 0, 0)),
        out_shape=jax.ShapeDtypeStruct((_B, 1, _E), jnp.float32),
        scratch_shapes=[pltpu.VMEM((_E, _N), jnp.float32)],
    )(spd, src, w1t, a2, W2)
    return out.reshape(_B, _J, _J, 1)
